# baseline (speedup 1.0000x reference)
import jax
from jax import lax
import jax.numpy as jnp
from jax.experimental import pallas as pl
from jax.experimental.pallas import tpu as pltpu
from jax.experimental.pallas import tpu_sc as plsc

_D = 256
_WIN = 64
_NBUF = 3


def _sc_gather_t(W, idx_flat, t, b):
    n = idx_flat.shape[0]
    nw = 32
    wpw = n // (_WIN * nw)
    jmax = b // _WIN
    mesh = plsc.VectorSubcoreMesh(core_axis_name="core",
                                  subcore_axis_name="subcore")

    @pl.kernel(
        out_type=jax.ShapeDtypeStruct((t, b, _D), jnp.float32),
        mesh=mesh,
        scratch_types=(
            [pltpu.VMEM((wpw * _WIN,), jnp.int32),
             pltpu.VMEM((W.shape[0], _D), jnp.float32),
             pltpu.VMEM((_NBUF, _WIN, _D), jnp.float32)]
            + [pltpu.SemaphoreType.DMA] * (2 * _NBUF)
        ),
    )
    def k(w_hbm, i_hbm, o_hbm, idx_v, w_v, rows_v, *sems):
        gsem = sems[:_NBUF]
        wsem = sems[_NBUF:]
        wid = lax.axis_index("subcore") * 2 + lax.axis_index("core")
        base = wid * wpw

        pltpu.sync_copy(i_hbm.at[pl.ds(base * _WIN, wpw * _WIN)], idx_v)
        pltpu.sync_copy(w_hbm, w_v)

        def gather(k_, p):
            return pltpu.make_async_copy(
                w_hbm.at[idx_v.at[pl.ds(k_ * _WIN, _WIN)]],
                rows_v.at[p], gsem[p])

        def write(k_, p):
            m = base + k_
            return pltpu.make_async_copy(
                rows_v.at[p],
                o_hbm.at[m // jmax, pl.ds((m % jmax) * _WIN, _WIN)],
                wsem[p])

        def assemble(k_, p):
            for g in range(_WIN // 16):
                idvec = idx_v[pl.ds(k_ * _WIN + g * 16, 16)]
                for li in range(16):
                    rid = idvec[li]
                    r = g * 16 + li
                    for j in range(_D // 16):
                        rows_v[p, r, pl.ds(j * 16, 16)] = (
                            w_v[rid, pl.ds(j * 16, 16)])

        gather(0, 0).start()

        @pl.loop(0, wpw, step=4)
        def _(k0):
            gather(k0, 0).wait()
            write(k0, 0).start()

            @pl.when(k0 >= 2)
            def _():
                write(k0 - 2, 1).wait()

            gather(k0 + 2, 1).start()

            @pl.when(k0 >= 1)
            def _():
                write(k0 - 1, 2).wait()

            assemble(k0 + 1, 2)
            write(k0 + 1, 2).start()

            gather(k0 + 2, 1).wait()
            write(k0 + 2, 1).start()
            write(k0, 0).wait()

            @pl.when(k0 + 4 < wpw)
            def _():
                gather(k0 + 4, 0).start()

            write(k0 + 1, 2).wait()
            assemble(k0 + 3, 2)
            write(k0 + 3, 2).start()

        write(wpw - 2, 1).wait()
        write(wpw - 1, 2).wait()

    return k(W, idx_flat)


def kernel(ids, W):
    b, t = ids.shape
    idx_t = ids.astype(jnp.int32).T
    idx_flat = idx_t.reshape(-1)
    out_t = _sc_gather_t(W, idx_flat, t, b)
    return out_t.transpose(1, 0, 2)


# speedup vs baseline: 1.2165x; 1.2165x over previous
import jax
from jax import lax
import jax.numpy as jnp
from jax.experimental import pallas as pl
from jax.experimental.pallas import tpu as pltpu
from jax.experimental.pallas import tpu_sc as plsc

_D = 256
_WIN = 64
_NBUF = 3


def _sc_gather_t(W, idx_flat, t, b):
    n = idx_flat.shape[0]
    nw = 32
    wpw = n // (_WIN * nw)
    jmax = b // _WIN
    mesh = plsc.VectorSubcoreMesh(core_axis_name="core",
                                  subcore_axis_name="subcore")

    @pl.kernel(
        out_type=jax.ShapeDtypeStruct((t, b, _D), jnp.float32),
        mesh=mesh,
        scratch_types=(
            [pltpu.VMEM((wpw * _WIN,), jnp.int32),
             pltpu.VMEM((W.shape[0], _D), jnp.float32),
             pltpu.VMEM((_NBUF, _WIN, _D), jnp.float32)]
            + [pltpu.SemaphoreType.DMA] * (2 * _NBUF)
        ),
    )
    def k(w_hbm, i_hbm, o_hbm, idx_v, w_v, rows_v, *sems):
        gsem = sems[:_NBUF]
        wsem = sems[_NBUF:]
        wid = lax.axis_index("subcore") * 2 + lax.axis_index("core")
        base = wid * wpw

        pltpu.sync_copy(i_hbm.at[pl.ds(base * _WIN, wpw * _WIN)], idx_v)
        pltpu.sync_copy(w_hbm, w_v)

        def gather(k_, p):
            return pltpu.make_async_copy(
                w_hbm.at[idx_v.at[pl.ds(k_ * _WIN, _WIN)]],
                rows_v.at[p], gsem[p])

        def write(k_, p):
            m = base + k_
            return pltpu.make_async_copy(
                rows_v.at[p],
                o_hbm.at[m // jmax, pl.ds((m % jmax) * _WIN, _WIN)],
                wsem[p])

        def assemble(k_, p):
            for g in range(_WIN // 16):
                idvec = idx_v[pl.ds(k_ * _WIN + g * 16, 16)]
                for li in range(16):
                    rid = idvec[li]
                    r = g * 16 + li
                    for j in range(_D // 16):
                        rows_v[p, r, pl.ds(j * 16, 16)] = (
                            w_v[rid, pl.ds(j * 16, 16)])

        gather(0, 0).start()
        gather(1, 1).start()

        @pl.loop(0, wpw - 1, step=_NBUF)
        def _(k0):
            gather(k0, 0).wait()
            write(k0, 0).start()

            @pl.when(k0 >= 1)
            def _():
                write(k0 - 1, 2).wait()

            gather(k0 + 1, 1).wait()
            write(k0 + 1, 1).start()
            write(k0, 0).wait()
            gather(k0 + 3, 0).start()

            assemble(k0 + 2, 2)
            write(k0 + 2, 2).start()
            write(k0 + 1, 1).wait()

            @pl.when(k0 + 4 < wpw)
            def _():
                gather(k0 + 4, 1).start()

        gather(wpw - 1, 0).wait()
        write(wpw - 1, 0).start()
        write(wpw - 2, 2).wait()
        write(wpw - 1, 0).wait()

    return k(W, idx_flat)


def kernel(ids, W):
    b, t = ids.shape
    idx_t = ids.astype(jnp.int32).T
    idx_flat = idx_t.reshape(-1)
    out_t = _sc_gather_t(W, idx_flat, t, b)
    return out_t.transpose(1, 0, 2)
